# Initial kernel scaffold; baseline (speedup 1.0000x reference)
import jax
import jax.numpy as jnp
from jax.experimental import pallas as pl


def kernel(y, x):
    raise NotImplementedError("write your pallas kernel here")


# speedup vs baseline: 3054.5074x; 3054.5074x over previous
import functools

import jax
import jax.numpy as jnp
import numpy as np
from jax import lax
from jax.experimental import pallas as pl
from jax.experimental.pallas import tpu as pltpu
from jax.experimental.pallas import tpu_sc as plsc

N_FFT = 2048
HOP = 512
NBINS = N_FFT // 2 + 1
NF = 1040
T = 257
TBLK = 32
NTBLK = 9
TPAD = NTBLK * TBLK
REGION = TBLK * HOP + (N_FFT - HOP)
LPAD = (NTBLK - 1) * TBLK * HOP + REGION
NW = 32
RPW = 65
NCHUNK = NF // 16
BSTEPS = 11


def _dft_consts():
    n = np.arange(N_FFT, dtype=np.float64)
    k = np.arange(NF, dtype=np.float64)
    ang = 2.0 * np.pi * np.outer(n, k) / N_FFT
    cos = np.cos(ang)
    sin = np.sin(ang)
    cos[:, NBINS:] = 0.0
    sin[:, NBINS:] = 0.0
    w = np.concatenate([cos, sin], axis=1).astype(np.float32)
    hann = (0.5 - 0.5 * np.cos(2.0 * np.pi * n / N_FFT)).astype(np.float32)
    tri = np.triu(np.ones((NF, NF), dtype=np.float32))
    return w, hann.reshape(1, N_FFT), tri


_W, _HANN, _TRI = _dft_consts()


def _stft_cdf_body(xy_ref, w_ref, hann_ref, tri_ref, M_ref, m_ref, cm_ref):
    g = pl.program_id(1)
    reg = xy_ref[:, 0, pl.ds(g * TBLK, TBLK + 3), :]
    frames = jnp.concatenate(
        [reg[:, j:j + TBLK, :] for j in range(4)], axis=2
    )
    frames = frames.reshape(2 * TBLK, N_FFT) * hann_ref[...]
    o = lax.dot(frames, w_ref[...], precision=lax.Precision.HIGHEST)
    c = o[:, :NF]
    s = o[:, NF:]
    mag = jnp.sqrt(c * c + s * s)
    F = lax.dot(mag, tri_ref[...], precision=lax.Precision.HIGHEST)
    Fn = F / F[:, NF - 1:NF]
    Fx = Fn[:TBLK]
    Fy = Fn[TBLK:]
    mm = jnp.minimum(Fx, Fy)
    M_ref[0] = jnp.maximum(Fx, Fy)
    m_ref[0] = mm
    cm_ref[0] = lax.dot(mm, tri_ref[...], precision=lax.Precision.HIGHEST)


def _stft_cdf(xy):
    bs = xy.shape[1]
    out_sds = jax.ShapeDtypeStruct((bs, TPAD, NF), jnp.float32)
    return pl.pallas_call(
        _stft_cdf_body,
        grid=(bs, NTBLK),
        in_specs=[
            pl.BlockSpec((2, 1, LPAD // 512, 512), lambda b, g: (0, b, 0, 0)),
            pl.BlockSpec((N_FFT, 2 * NF), lambda b, g: (0, 0)),
            pl.BlockSpec((1, N_FFT), lambda b, g: (0, 0)),
            pl.BlockSpec((NF, NF), lambda b, g: (0, 0)),
        ],
        out_specs=[
            pl.BlockSpec((1, TBLK, NF), lambda b, g: (b, g, 0)),
            pl.BlockSpec((1, TBLK, NF), lambda b, g: (b, g, 0)),
            pl.BlockSpec((1, TBLK, NF), lambda b, g: (b, g, 0)),
        ],
        out_shape=[out_sds, out_sds, out_sds],
    )(xy, jnp.asarray(_W), jnp.asarray(_HANN), jnp.asarray(_TRI))


def _ot_body(M_hbm, m_hbm, cm_hbm, out_hbm, Mv, mv, cmv, res):
    wid = lax.axis_index("s") * 2 + lax.axis_index("c")
    iot = lax.iota(jnp.int32, 16)
    nrows = M_hbm.shape[0] // TPAD * T

    def row_body(i, carry):
        k = wid * RPW + i

        @pl.when(k < nrows)
        def _():
            b = k // T
            t = k - b * T
            r = b * TPAD + t
            pltpu.sync_copy(M_hbm.at[r], Mv)
            pltpu.sync_copy(m_hbm.at[r], mv)
            pltpu.sync_copy(cm_hbm.at[r], cmv)

            def chunk_body(cc, acc):
                q = Mv[pl.ds(cc * 16, 16)]
                lo0 = jnp.zeros((16,), jnp.int32)
                hi0 = jnp.full((16,), NBINS, jnp.int32)

                def bs_body(_, lh):
                    lo, hi = lh
                    mid = (lo + hi) >> 1
                    mval = plsc.load_gather(mv, [mid])
                    pred = mval < q
                    return (jnp.where(pred, mid + 1, lo),
                            jnp.where(pred, hi, mid))

                K, _ = lax.fori_loop(0, BSTEPS, bs_body, (lo0, hi0))
                jvec = cc * 16 + iot
                kp = jnp.maximum(K, jvec + 1)
                cmk = plsc.load_gather(cmv, [kp - 1])
                cmj = cmv[pl.ds(cc * 16, 16)]
                mj = mv[pl.ds(cc * 16, 16)]
                cnt = (kp - 1 - jvec).astype(jnp.float32)
                return acc + (q - mj) + 2.0 * (cnt * q - cmk + cmj)

            acc = lax.fori_loop(0, NCHUNK, chunk_body,
                                jnp.zeros((16,), jnp.float32))
            sval = jnp.sum(acc)
            lane = i & 15
            plsc.store_scatter(res, [jnp.full((16,), i, jnp.int32)],
                               jnp.full((16,), sval, jnp.float32),
                               mask=iot == lane)

        return carry

    lax.fori_loop(0, RPW, row_body, 0)
    pltpu.sync_copy(res, out_hbm.at[wid])


_ot_sc = functools.partial(
    pl.kernel,
    out_type=jax.ShapeDtypeStruct((NW, 80), jnp.float32),
    mesh=plsc.VectorSubcoreMesh(core_axis_name="c", subcore_axis_name="s"),
    scratch_types=[
        pltpu.VMEM((NF,), jnp.float32),
        pltpu.VMEM((NF,), jnp.float32),
        pltpu.VMEM((NF,), jnp.float32),
        pltpu.VMEM((80,), jnp.float32),
    ],
    compiler_params=pltpu.CompilerParams(needs_layout_passes=False),
)(_ot_body)


@jax.jit
def kernel(y, x):
    bs = x.shape[0]
    pad = N_FFT // 2
    xp = jnp.pad(x, ((0, 0), (pad, pad)), mode="reflect")
    yp = jnp.pad(y, ((0, 0), (pad, pad)), mode="reflect")
    xy = jnp.stack([xp, yp])
    xy = jnp.pad(xy, ((0, 0), (0, 0), (0, LPAD - xy.shape[2])))
    xy = xy.reshape(2, bs, LPAD // 512, 512)
    M3, m3, cm3 = _stft_cdf(xy)
    nrows = bs * T
    S = _ot_sc(M3.reshape(bs * TPAD, NF), m3.reshape(bs * TPAD, NF),
               cm3.reshape(bs * TPAD, NF))
    S = S[:, :RPW].reshape(-1)[:nrows]
    d = 1.0 / N_FFT
    return S.reshape(bs, T).mean(axis=1) * (100.0 * d * d)


# speedup vs baseline: 5733.6145x; 1.8771x over previous
import functools

import jax
import jax.numpy as jnp
import numpy as np
from jax import lax
from jax.experimental import pallas as pl
from jax.experimental.pallas import tpu as pltpu
from jax.experimental.pallas import tpu_sc as plsc

N_FFT = 2048
HOP = 512
NBINS = N_FFT // 2 + 1
NF = 1040
T = 257
TBLK = 32
NTBLK = 9
TPAD = NTBLK * TBLK
REGION = TBLK * HOP + (N_FFT - HOP)
LPAD = (NTBLK - 1) * TBLK * HOP + REGION
NW = 32
RPW = 65
NCHUNK = NF // 16
BSTEPS = 11


def _dft_consts():
    n = np.arange(N_FFT, dtype=np.float64)
    k = np.arange(NF, dtype=np.float64)
    ang = 2.0 * np.pi * np.outer(n, k) / N_FFT
    cos = np.cos(ang)
    sin = np.sin(ang)
    cos[:, NBINS:] = 0.0
    sin[:, NBINS:] = 0.0
    w = np.concatenate([cos, sin], axis=1).astype(np.float32)
    wh = w.astype(jnp.bfloat16)
    wl = (w - wh.astype(np.float32)).astype(jnp.bfloat16)
    hann = (0.5 - 0.5 * np.cos(2.0 * np.pi * n / N_FFT)).astype(np.float32)
    tri = np.triu(np.ones((NF, NF), dtype=np.float32))
    return wh, wl, hann.reshape(1, N_FFT), tri.astype(jnp.bfloat16)


_WH, _WL, _HANN, _TRI = _dft_consts()


def _split_dot(a, bh, bl=None):
    ah = a.astype(jnp.bfloat16)
    al = (a - ah.astype(jnp.float32)).astype(jnp.bfloat16)
    f32 = jnp.float32
    o = jnp.dot(ah, bh, preferred_element_type=f32)
    o += jnp.dot(al, bh, preferred_element_type=f32)
    if bl is not None:
        o += jnp.dot(ah, bl, preferred_element_type=f32)
    return o


def _stft_cdf_body(xy_ref, wh_ref, wl_ref, hann_ref, tri_ref,
                   M_ref, m_ref, cm_ref):
    g = pl.program_id(1)
    reg = xy_ref[:, 0, pl.ds(g * TBLK, TBLK + 3), :]
    frames = jnp.concatenate(
        [reg[:, j:j + TBLK, :] for j in range(4)], axis=2
    )
    frames = frames.reshape(2 * TBLK, N_FFT) * hann_ref[...]
    o = _split_dot(frames, wh_ref[...], wl_ref[...])
    c = o[:, :NF]
    s = o[:, NF:]
    mag = jnp.sqrt(c * c + s * s)
    F = _split_dot(mag, tri_ref[...])
    Fn = F / F[:, NF - 1:NF]
    Fx = Fn[:TBLK]
    Fy = Fn[TBLK:]
    mm = jnp.minimum(Fx, Fy)
    M_ref[0] = jnp.maximum(Fx, Fy)
    m_ref[0] = mm
    cm_ref[0] = _split_dot(mm, tri_ref[...])


def _stft_cdf(xy):
    bs = xy.shape[1]
    out_sds = jax.ShapeDtypeStruct((bs, TPAD, NF), jnp.float32)
    return pl.pallas_call(
        _stft_cdf_body,
        grid=(bs, NTBLK),
        in_specs=[
            pl.BlockSpec((2, 1, LPAD // 512, 512), lambda b, g: (0, b, 0, 0)),
            pl.BlockSpec((N_FFT, 2 * NF), lambda b, g: (0, 0)),
            pl.BlockSpec((N_FFT, 2 * NF), lambda b, g: (0, 0)),
            pl.BlockSpec((1, N_FFT), lambda b, g: (0, 0)),
            pl.BlockSpec((NF, NF), lambda b, g: (0, 0)),
        ],
        out_specs=[
            pl.BlockSpec((1, TBLK, NF), lambda b, g: (b, g, 0)),
            pl.BlockSpec((1, TBLK, NF), lambda b, g: (b, g, 0)),
            pl.BlockSpec((1, TBLK, NF), lambda b, g: (b, g, 0)),
        ],
        out_shape=[out_sds, out_sds, out_sds],
    )(xy, jnp.asarray(_WH), jnp.asarray(_WL), jnp.asarray(_HANN),
      jnp.asarray(_TRI))


def _ot_body(M_hbm, m_hbm, cm_hbm, out_hbm, Mv, mv, cmv, res):
    wid = lax.axis_index("s") * 2 + lax.axis_index("c")
    iot = lax.iota(jnp.int32, 16)
    nrows = M_hbm.shape[0] // TPAD * T

    def row_body(i, carry):
        k = wid * RPW + i

        @pl.when(k < nrows)
        def _():
            b = k // T
            t = k - b * T
            r = b * TPAD + t
            pltpu.sync_copy(M_hbm.at[r], Mv)
            pltpu.sync_copy(m_hbm.at[r], mv)
            pltpu.sync_copy(cm_hbm.at[r], cmv)

            def chunk_body(cc, acc):
                q = Mv[pl.ds(cc * 16, 16)]
                lo0 = jnp.zeros((16,), jnp.int32)
                hi0 = jnp.full((16,), NBINS, jnp.int32)

                def bs_body(_, lh):
                    lo, hi = lh
                    mid = (lo + hi) >> 1
                    mval = plsc.load_gather(mv, [mid])
                    pred = mval < q
                    return (jnp.where(pred, mid + 1, lo),
                            jnp.where(pred, hi, mid))

                K, _ = lax.fori_loop(0, BSTEPS, bs_body, (lo0, hi0))
                jvec = cc * 16 + iot
                kp = jnp.maximum(K, jvec + 1)
                cmk = plsc.load_gather(cmv, [kp - 1])
                cmj = cmv[pl.ds(cc * 16, 16)]
                mj = mv[pl.ds(cc * 16, 16)]
                cnt = (kp - 1 - jvec).astype(jnp.float32)
                return acc + (q - mj) + 2.0 * (cnt * q - cmk + cmj)

            acc = lax.fori_loop(0, NCHUNK, chunk_body,
                                jnp.zeros((16,), jnp.float32))
            sval = jnp.sum(acc)
            lane = i & 15
            plsc.store_scatter(res, [jnp.full((16,), i, jnp.int32)],
                               jnp.full((16,), sval, jnp.float32),
                               mask=iot == lane)

        return carry

    lax.fori_loop(0, RPW, row_body, 0)
    pltpu.sync_copy(res, out_hbm.at[wid])


_ot_sc = functools.partial(
    pl.kernel,
    out_type=jax.ShapeDtypeStruct((NW, 80), jnp.float32),
    mesh=plsc.VectorSubcoreMesh(core_axis_name="c", subcore_axis_name="s"),
    scratch_types=[
        pltpu.VMEM((NF,), jnp.float32),
        pltpu.VMEM((NF,), jnp.float32),
        pltpu.VMEM((NF,), jnp.float32),
        pltpu.VMEM((80,), jnp.float32),
    ],
    compiler_params=pltpu.CompilerParams(needs_layout_passes=False),
)(_ot_body)


@jax.jit
def kernel(y, x):
    bs = x.shape[0]
    pad = N_FFT // 2
    xp = jnp.pad(x, ((0, 0), (pad, pad)), mode="reflect")
    yp = jnp.pad(y, ((0, 0), (pad, pad)), mode="reflect")
    xy = jnp.stack([xp, yp])
    xy = jnp.pad(xy, ((0, 0), (0, 0), (0, LPAD - xy.shape[2])))
    xy = xy.reshape(2, bs, LPAD // 512, 512)
    M3, m3, cm3 = _stft_cdf(xy)
    nrows = bs * T
    S = _ot_sc(M3.reshape(bs * TPAD, NF), m3.reshape(bs * TPAD, NF),
               cm3.reshape(bs * TPAD, NF))
    S = S[:, :RPW].reshape(-1)[:nrows]
    d = 1.0 / N_FFT
    return S.reshape(bs, T).mean(axis=1) * (100.0 * d * d)


# speedup vs baseline: 11033.4041x; 1.9243x over previous
import functools

import jax
import jax.numpy as jnp
import numpy as np
from jax import lax
from jax.experimental import pallas as pl
from jax.experimental.pallas import tpu as pltpu
from jax.experimental.pallas import tpu_sc as plsc

N_FFT = 2048
HOP = 512
NBINS = N_FFT // 2 + 1
NF = 1040
T = 257
TBLK = 32
NTBLK = 9
TPAD = NTBLK * TBLK
REGION = TBLK * HOP + (N_FFT - HOP)
LPAD = (NTBLK - 1) * TBLK * HOP + REGION
NW = 32
RPW = 65
NCHUNK = NF // 16
BSTEPS = 11


def _dft_consts():
    n = np.arange(N_FFT, dtype=np.float64)
    k = np.arange(NF, dtype=np.float64)
    ang = 2.0 * np.pi * np.outer(n, k) / N_FFT
    cos = np.cos(ang)
    sin = np.sin(ang)
    cos[:, NBINS:] = 0.0
    sin[:, NBINS:] = 0.0
    w = np.concatenate([cos, sin], axis=1).astype(np.float32)
    wh = w.astype(jnp.bfloat16)
    wl = (w - wh.astype(np.float32)).astype(jnp.bfloat16)
    hann = (0.5 - 0.5 * np.cos(2.0 * np.pi * n / N_FFT)).astype(np.float32)
    tri = np.triu(np.ones((NF, NF), dtype=np.float32))
    return wh, wl, hann.reshape(1, N_FFT), tri.astype(jnp.bfloat16)


_WH, _WL, _HANN, _TRI = _dft_consts()


def _split_dot(a, bh, bl=None):
    ah = a.astype(jnp.bfloat16)
    al = (a - ah.astype(jnp.float32)).astype(jnp.bfloat16)
    f32 = jnp.float32
    o = jnp.dot(ah, bh, preferred_element_type=f32)
    o += jnp.dot(al, bh, preferred_element_type=f32)
    if bl is not None:
        o += jnp.dot(ah, bl, preferred_element_type=f32)
    return o


def _stft_cdf_body(xy_ref, wh_ref, wl_ref, hann_ref, tri_ref, out_ref):
    g = pl.program_id(0)
    bs = xy_ref.shape[1]
    rows = 2 * bs * TBLK
    reg = xy_ref[:, :, pl.ds(g * TBLK, TBLK + 3), :]
    frames = jnp.concatenate(
        [reg[:, :, j:j + TBLK, :] for j in range(4)], axis=3
    )
    frames = frames.reshape(rows, N_FFT) * hann_ref[...]
    o = _split_dot(frames, wh_ref[...], wl_ref[...])
    c = o[:, :NF]
    s = o[:, NF:]
    mag = jnp.sqrt(c * c + s * s)
    F = _split_dot(mag, tri_ref[...])
    Fn = F / F[:, NF - 1:NF]
    Fx = Fn[:rows // 2]
    Fy = Fn[rows // 2:]
    mm = jnp.minimum(Fx, Fy)
    cm = _split_dot(mm, tri_ref[...])
    out_ref[:, :, 0:NF] = jnp.maximum(Fx, Fy).reshape(bs, TBLK, NF)
    out_ref[:, :, NF:2 * NF] = mm.reshape(bs, TBLK, NF)
    out_ref[:, :, 2 * NF:3 * NF] = cm.reshape(bs, TBLK, NF)


def _stft_cdf(xy):
    bs = xy.shape[1]
    out_sds = jax.ShapeDtypeStruct((bs, TPAD, 3 * NF), jnp.float32)
    return pl.pallas_call(
        _stft_cdf_body,
        grid=(NTBLK,),
        in_specs=[
            pl.BlockSpec((2, bs, LPAD // 512, 512), lambda g: (0, 0, 0, 0)),
            pl.BlockSpec((N_FFT, 2 * NF), lambda g: (0, 0)),
            pl.BlockSpec((N_FFT, 2 * NF), lambda g: (0, 0)),
            pl.BlockSpec((1, N_FFT), lambda g: (0, 0)),
            pl.BlockSpec((NF, NF), lambda g: (0, 0)),
        ],
        out_specs=pl.BlockSpec((bs, TBLK, 3 * NF), lambda g: (0, g, 0)),
        out_shape=out_sds,
    )(xy, jnp.asarray(_WH), jnp.asarray(_WL), jnp.asarray(_HANN),
      jnp.asarray(_TRI))


def _ot_body(P_hbm, out_hbm, pa, pb, res, sem):
    wid = lax.axis_index("s") * 2 + lax.axis_index("c")
    iot = lax.iota(jnp.int32, 16)
    nrows = P_hbm.shape[0] // TPAD * T
    base = wid * RPW

    def rk(k):
        kc = jnp.minimum(k, nrows - 1)
        b = kc // T
        return b * TPAD + (kc - b * T)

    def compute(buf, i):
        def chunk_body(cc, acc):
            q = buf[pl.ds(cc * 16, 16)]
            lo0 = jnp.zeros((16,), jnp.int32)
            hi0 = jnp.full((16,), NBINS, jnp.int32)

            def bs_body(_, lh):
                lo, hi = lh
                mid = (lo + hi) >> 1
                mval = plsc.load_gather(buf, [NF + mid])
                pred = mval < q
                return (jnp.where(pred, mid + 1, lo),
                        jnp.where(pred, hi, mid))

            K, _ = lax.fori_loop(0, BSTEPS, bs_body, (lo0, hi0))
            jvec = cc * 16 + iot
            kp = jnp.maximum(K, jvec + 1)
            cmk = plsc.load_gather(buf, [2 * NF + kp - 1])
            cmj = buf[pl.ds(2 * NF + cc * 16, 16)]
            mj = buf[pl.ds(NF + cc * 16, 16)]
            cnt = (kp - 1 - jvec).astype(jnp.float32)
            return acc + (q - mj) + 2.0 * (cnt * q - cmk + cmj)

        acc = lax.fori_loop(0, NCHUNK, chunk_body,
                            jnp.zeros((16,), jnp.float32))
        sval = jnp.sum(acc)
        plsc.store_scatter(res, [jnp.full((16,), i, jnp.int32)],
                           jnp.full((16,), sval, jnp.float32),
                           mask=iot == (i & 15))

    pltpu.async_copy(P_hbm.at[rk(base)], pa, sem)

    def pair_body(p, carry):
        i0 = p * 2
        pltpu.make_async_copy(P_hbm.at[rk(base)], pa, sem).wait()
        pltpu.async_copy(P_hbm.at[rk(base + i0 + 1)], pb, sem)
        compute(pa, i0)
        pltpu.make_async_copy(P_hbm.at[rk(base)], pb, sem).wait()
        pltpu.async_copy(P_hbm.at[rk(base + i0 + 2)], pa, sem)
        compute(pb, i0 + 1)
        return carry

    lax.fori_loop(0, RPW // 2, pair_body, 0)
    pltpu.make_async_copy(P_hbm.at[rk(base)], pa, sem).wait()
    compute(pa, RPW - 1)
    pltpu.sync_copy(res, out_hbm.at[wid])


_ot_sc = functools.partial(
    pl.kernel,
    out_type=jax.ShapeDtypeStruct((NW, 80), jnp.float32),
    mesh=plsc.VectorSubcoreMesh(core_axis_name="c", subcore_axis_name="s"),
    scratch_types=[
        pltpu.VMEM((3 * NF,), jnp.float32),
        pltpu.VMEM((3 * NF,), jnp.float32),
        pltpu.VMEM((80,), jnp.float32),
        pltpu.SemaphoreType.DMA,
    ],
    compiler_params=pltpu.CompilerParams(needs_layout_passes=False),
)(_ot_body)


@jax.jit
def kernel(y, x):
    bs = x.shape[0]
    pad = N_FFT // 2
    xp = jnp.pad(x, ((0, 0), (pad, pad)), mode="reflect")
    yp = jnp.pad(y, ((0, 0), (pad, pad)), mode="reflect")
    xy = jnp.stack([xp, yp])
    xy = jnp.pad(xy, ((0, 0), (0, 0), (0, LPAD - xy.shape[2])))
    xy = xy.reshape(2, bs, LPAD // 512, 512)
    P = _stft_cdf(xy)
    nrows = bs * T
    S = _ot_sc(P.reshape(bs * TPAD, 3 * NF))
    S = S[:, :RPW].reshape(-1)[:nrows]
    d = 1.0 / N_FFT
    return S.reshape(bs, T).mean(axis=1) * (100.0 * d * d)


# speedup vs baseline: 13874.1546x; 1.2575x over previous
import functools

import jax
import jax.numpy as jnp
import numpy as np
from jax import lax
from jax.experimental import pallas as pl
from jax.experimental.pallas import tpu as pltpu
from jax.experimental.pallas import tpu_sc as plsc

N_FFT = 2048
HOP = 512
NBINS = N_FFT // 2 + 1
NF = 1040
T = 257
TBLK = 32
NTBLK = 9
TPAD = NTBLK * TBLK
REGION = TBLK * HOP + (N_FFT - HOP)
LPAD = (NTBLK - 1) * TBLK * HOP + REGION
NW = 32
RPW = 65
NCHUNK = NF // 16
BSTEPS = 11


def _dft_consts():
    n = np.arange(N_FFT, dtype=np.float64)
    k = np.arange(NF, dtype=np.float64)
    ang = 2.0 * np.pi * np.outer(n, k) / N_FFT
    cos = np.cos(ang)
    sin = np.sin(ang)
    cos[:, NBINS:] = 0.0
    sin[:, NBINS:] = 0.0
    w = np.concatenate([cos, sin], axis=1).astype(np.float32)
    wh = w.astype(jnp.bfloat16)
    wl = (w - wh.astype(np.float32)).astype(jnp.bfloat16)
    hann = (0.5 - 0.5 * np.cos(2.0 * np.pi * n / N_FFT)).astype(np.float32)
    tri = np.triu(np.ones((NF, NF), dtype=np.float32))
    return wh, wl, hann.reshape(1, N_FFT), tri.astype(jnp.bfloat16)


_WH, _WL, _HANN, _TRI = _dft_consts()


def _split_dot(a, bh, bl=None):
    ah = a.astype(jnp.bfloat16)
    al = (a - ah.astype(jnp.float32)).astype(jnp.bfloat16)
    f32 = jnp.float32
    o = jnp.dot(ah, bh, preferred_element_type=f32)
    o += jnp.dot(al, bh, preferred_element_type=f32)
    if bl is not None:
        o += jnp.dot(ah, bl, preferred_element_type=f32)
    return o


def _stft_cdf_body(xy_ref, wh_ref, wl_ref, hann_ref, tri_ref, out_ref):
    g = pl.program_id(0)
    bs = xy_ref.shape[1]
    rows = 2 * bs * TBLK
    reg = xy_ref[:, :, pl.ds(g * TBLK, TBLK + 3), :]
    frames = jnp.concatenate(
        [reg[:, :, j:j + TBLK, :] for j in range(4)], axis=3
    )
    frames = frames.reshape(rows, N_FFT) * hann_ref[...]
    o = _split_dot(frames, wh_ref[...], wl_ref[...])
    c = o[:, :NF]
    s = o[:, NF:]
    mag = jnp.sqrt(c * c + s * s)
    F = _split_dot(mag, tri_ref[...])
    Fn = F / F[:, NF - 1:NF]
    Fx = Fn[:rows // 2]
    Fy = Fn[rows // 2:]
    mm = jnp.minimum(Fx, Fy)
    cm = _split_dot(mm, tri_ref[...])
    out_ref[:, :, 0:NF] = jnp.maximum(Fx, Fy).reshape(bs, TBLK, NF)
    out_ref[:, :, NF:2 * NF] = mm.reshape(bs, TBLK, NF)
    out_ref[:, :, 2 * NF:3 * NF] = cm.reshape(bs, TBLK, NF)


def _stft_cdf(xy):
    bs = xy.shape[1]
    out_sds = jax.ShapeDtypeStruct((bs, TPAD, 3 * NF), jnp.float32)
    return pl.pallas_call(
        _stft_cdf_body,
        grid=(NTBLK,),
        in_specs=[
            pl.BlockSpec((2, bs, LPAD // 512, 512), lambda g: (0, 0, 0, 0)),
            pl.BlockSpec((N_FFT, 2 * NF), lambda g: (0, 0)),
            pl.BlockSpec((N_FFT, 2 * NF), lambda g: (0, 0)),
            pl.BlockSpec((1, N_FFT), lambda g: (0, 0)),
            pl.BlockSpec((NF, NF), lambda g: (0, 0)),
        ],
        out_specs=pl.BlockSpec((bs, TBLK, 3 * NF), lambda g: (0, g, 0)),
        out_shape=out_sds,
    )(xy, jnp.asarray(_WH), jnp.asarray(_WL), jnp.asarray(_HANN),
      jnp.asarray(_TRI))


def _ot_body(P_hbm, out_hbm, pa, pb, res, sem):
    wid = lax.axis_index("s") * 2 + lax.axis_index("c")
    iot = lax.iota(jnp.int32, 16)
    nrows = P_hbm.shape[0] // TPAD * T
    rpw = -(-nrows // NW)
    base = wid * rpw

    def rk(k):
        kc = jnp.minimum(k, nrows - 1)
        b = kc // T
        return b * TPAD + (kc - b * T)

    def compute(buf, i):
        def chunk_body(cc, acc):
            q = buf[pl.ds(cc * 16, 16)]
            lo0 = jnp.zeros((16,), jnp.int32)
            hi0 = jnp.full((16,), NBINS, jnp.int32)

            def bs_body(_, lh):
                lo, hi = lh
                mid = (lo + hi) >> 1
                mval = plsc.load_gather(buf, [NF + mid])
                pred = mval < q
                return (jnp.where(pred, mid + 1, lo),
                        jnp.where(pred, hi, mid))

            K, _ = lax.fori_loop(0, BSTEPS, bs_body, (lo0, hi0))
            jvec = cc * 16 + iot
            kp = jnp.maximum(K, jvec + 1)
            cmk = plsc.load_gather(buf, [2 * NF + kp - 1])
            cmj = buf[pl.ds(2 * NF + cc * 16, 16)]
            mj = buf[pl.ds(NF + cc * 16, 16)]
            cnt = (kp - 1 - jvec).astype(jnp.float32)
            return acc + (q - mj) + 2.0 * (cnt * q - cmk + cmj)

        acc = lax.fori_loop(0, NCHUNK, chunk_body,
                            jnp.zeros((16,), jnp.float32))
        sval = jnp.sum(acc)
        plsc.store_scatter(res, [jnp.full((16,), i, jnp.int32)],
                           jnp.full((16,), sval, jnp.float32),
                           mask=iot == (i & 15))

    pltpu.async_copy(P_hbm.at[rk(base)], pa, sem)

    def pair_body(p, carry):
        i0 = p * 2
        pltpu.make_async_copy(P_hbm.at[rk(base)], pa, sem).wait()
        pltpu.async_copy(P_hbm.at[rk(base + i0 + 1)], pb, sem)
        compute(pa, i0)
        pltpu.make_async_copy(P_hbm.at[rk(base)], pb, sem).wait()
        pltpu.async_copy(P_hbm.at[rk(base + i0 + 2)], pa, sem)
        compute(pb, i0 + 1)
        return carry

    lax.fori_loop(0, rpw // 2, pair_body, 0)
    pltpu.make_async_copy(P_hbm.at[rk(base)], pa, sem).wait()
    compute(pa, rpw - 1)
    pltpu.sync_copy(res, out_hbm.at[wid])


@functools.lru_cache(maxsize=None)
def _make_ot_sc(nrows):
    rpw = -(-nrows // NW)
    width = -(-rpw // 16) * 16
    return functools.partial(
        pl.kernel,
        out_type=jax.ShapeDtypeStruct((NW, width), jnp.float32),
        mesh=plsc.VectorSubcoreMesh(core_axis_name="c", subcore_axis_name="s"),
        scratch_types=[
            pltpu.VMEM((3 * NF,), jnp.float32),
            pltpu.VMEM((3 * NF,), jnp.float32),
            pltpu.VMEM((width,), jnp.float32),
            pltpu.SemaphoreType.DMA,
        ],
        compiler_params=pltpu.CompilerParams(needs_layout_passes=False),
    )(_ot_body)


@jax.jit
def kernel(y, x):
    bs = x.shape[0]
    pad = N_FFT // 2
    xp = jnp.pad(x, ((0, 0), (pad, pad)), mode="reflect")
    yp = jnp.pad(y, ((0, 0), (pad, pad)), mode="reflect")
    xy = jnp.stack([xp, yp])
    xy = jnp.pad(xy, ((0, 0), (0, 0), (0, LPAD - xy.shape[2])))
    xy = xy.reshape(2, bs, LPAD // 512, 512)
    hb = bs // 2
    parts = []
    for h in range(2):
        P = _stft_cdf(xy[:, h * hb:(h + 1) * hb])
        nrows = hb * T
        rpw = -(-nrows // NW)
        S = _make_ot_sc(nrows)(P.reshape(hb * TPAD, 3 * NF))
        parts.append(S[:, :rpw].reshape(-1)[:nrows].reshape(hb, T))
    d = 1.0 / N_FFT
    return jnp.concatenate(parts).mean(axis=1) * (100.0 * d * d)
